# baseline (speedup 1.0000x reference)
import functools

import jax
import jax.numpy as jnp
from jax import lax
from jax.experimental import pallas as pl
from jax.experimental.pallas import tpu as pltpu
from jax.experimental.pallas import tpu_sc as plsc

N_NODES = 10000
N_EDGES = 160000
NPAD = 10240
EPAD = 163840
D_IN, H, BOTTLE, DEC_H = 256, 512, 128, 512

NC, NS = 2, 16
CHK = 128
EPT = EPAD // NS
ITERS = EPT // CHK
CHKA = 64
ITERSA = EPT // CHKA
HALF_I = ITERSA // 2
ROWS_PT = NPAD // NS

_f32 = jnp.float32

_MESH = dict(core_axis_name="c", subcore_axis_name="s")


DEGW = 128


def _sc_degree(dst3, ones_hbm, zeros_hbm):
    @functools.partial(
        pl.kernel,
        out_type=jax.ShapeDtypeStruct((NPAD, DEGW), _f32),
        mesh=plsc.VectorSubcoreMesh(**_MESH),
        scratch_types=[
            pltpu.VMEM((ITERS, CHK), jnp.int32),
            pltpu.VMEM((CHK, DEGW), _f32),
            pltpu.VMEM_SHARED((NPAD, DEGW), _f32),
        ],
    )
    def deg_kernel(dst_hbm, ones_in, zeros_in, out_hbm, didx, ones_v, accum):
        c = lax.axis_index("c")
        s = lax.axis_index("s")
        row0 = s * ROWS_PT

        @pl.when(c == 0)
        def _():
            pltpu.sync_copy(zeros_in.at[pl.ds(row0, ROWS_PT)],
                            accum.at[pl.ds(row0, ROWS_PT)])
            pltpu.sync_copy(dst_hbm.at[s], didx)
            pltpu.sync_copy(ones_in, ones_v)
            plsc.subcore_barrier()

            def body(it, carry):
                pltpu.sync_copy(ones_v, accum.at[didx.at[it]], add=True)
                return carry

            lax.fori_loop(0, ITERS, body, 0)
            plsc.subcore_barrier()
            pltpu.sync_copy(accum.at[pl.ds(row0, ROWS_PT)],
                            out_hbm.at[pl.ds(row0, ROWS_PT)])

    return deg_kernel(dst3, ones_hbm, zeros_hbm)


def _gather_scatter_pipelined(g_ref, sidx, didx, accum, r0, r1, sem0, sem1,
                              base, iters):
    pltpu.async_copy(g_ref.at[sidx.at[base]], r0, sem0)

    def body(i2, carry):
        it = base + 2 * i2
        pltpu.async_copy(g_ref.at[sidx.at[it + 1]], r1, sem1)
        pltpu.make_async_copy(g_ref.at[sidx.at[it]], r0, sem0).wait()
        pltpu.sync_copy(r0, accum.at[didx.at[it]], add=True)

        @pl.when(2 * i2 + 2 < iters)
        def _():
            pltpu.async_copy(g_ref.at[sidx.at[it + 2]], r0, sem0)

        pltpu.make_async_copy(g_ref.at[sidx.at[it + 1]], r1, sem1).wait()
        pltpu.sync_copy(r1, accum.at[didx.at[it + 1]], add=True)
        return carry

    lax.fori_loop(0, iters // 2, body, 0)


def _agg_body(g_refs, src_hbm, dst_hbm, zeros_hbm, out_refs,
              sidx, didx, r0, r1, accum, sem0, sem1, cps):
    c = lax.axis_index("c")
    s = lax.axis_index("s")
    row0 = s * ROWS_PT
    for j in range(cps):
        pltpu.sync_copy(zeros_hbm.at[pl.ds(row0, ROWS_PT)],
                        accum.at[pl.ds(row0, ROWS_PT)])
        plsc.subcore_barrier()
        for cc in range(NC):
            cid = cc * cps + j

            @pl.when(c == cc)
            def _(cid=cid):
                for stage in range(2):
                    pltpu.sync_copy(src_hbm.at[s * 2 + stage], sidx)
                    pltpu.sync_copy(dst_hbm.at[s * 2 + stage], didx)
                    _gather_scatter_pipelined(g_refs[cid], sidx, didx, accum,
                                              r0, r1, sem0, sem1, 0, HALF_I)

        plsc.subcore_barrier()
        for cc in range(NC):
            cid = cc * cps + j

            @pl.when(c == cc)
            def _(cid=cid):
                pltpu.sync_copy(accum.at[pl.ds(row0, ROWS_PT)],
                                out_refs[cid].at[pl.ds(row0, ROWS_PT)])


def _sc_aggregate(g_chunks, src3, dst3, zeros_hbm, cw):
    nchunks = len(g_chunks)
    cps = nchunks // NC
    out_type = [jax.ShapeDtypeStruct((NPAD, cw), _f32)] * nchunks
    scratch = [
        pltpu.VMEM((HALF_I, CHKA), jnp.int32),
        pltpu.VMEM((HALF_I, CHKA), jnp.int32),
        pltpu.VMEM((CHKA, cw), _f32),
        pltpu.VMEM((CHKA, cw), _f32),
        pltpu.VMEM_SHARED((NPAD, cw), _f32),
        pltpu.SemaphoreType.DMA,
        pltpu.SemaphoreType.DMA,
    ]
    mesh = plsc.VectorSubcoreMesh(**_MESH)

    if nchunks == 4:
        @functools.partial(pl.kernel, out_type=out_type, mesh=mesh,
                           scratch_types=scratch)
        def agg4(g0, g1, g2, g3, src_hbm, dst_hbm, zin, o0, o1, o2, o3,
                 sidx, didx, r0, r1, accum, sem0, sem1):
            _agg_body((g0, g1, g2, g3), src_hbm, dst_hbm, zin,
                      (o0, o1, o2, o3), sidx, didx, r0, r1, accum,
                      sem0, sem1, cps)

        return agg4(*g_chunks, src3, dst3, zeros_hbm)

    @functools.partial(pl.kernel, out_type=out_type, mesh=mesh,
                       scratch_types=scratch)
    def agg2(g0, g1, src_hbm, dst_hbm, zin, o0, o1,
             sidx, didx, r0, r1, accum, sem0, sem1):
        _agg_body((g0, g1), src_hbm, dst_hbm, zin, (o0, o1),
                  sidx, didx, r0, r1, accum, sem0, sem1, cps)

    return agg2(*g_chunks, src3, dst3, zeros_hbm)


def _sc_aggregate_split(g, src3, dst3, zeros_hbm):
    cw = g.shape[1]
    out_type = [jax.ShapeDtypeStruct((NPAD, cw), _f32)] * NC

    @functools.partial(
        pl.kernel,
        out_type=out_type,
        mesh=plsc.VectorSubcoreMesh(**_MESH),
        scratch_types=[
            pltpu.VMEM((HALF_I, CHKA), jnp.int32),
            pltpu.VMEM((HALF_I, CHKA), jnp.int32),
            pltpu.VMEM((CHKA, cw), _f32),
            pltpu.VMEM((CHKA, cw), _f32),
            pltpu.VMEM_SHARED((NPAD, cw), _f32),
            pltpu.SemaphoreType.DMA,
            pltpu.SemaphoreType.DMA,
        ],
    )
    def aggs(g_hbm, src_hbm, dst_hbm, zin, o0, o1,
             sidx, didx, r0, r1, accum, sem0, sem1):
        c = lax.axis_index("c")
        s = lax.axis_index("s")
        row0 = s * ROWS_PT
        pltpu.sync_copy(zin.at[pl.ds(row0, ROWS_PT)],
                        accum.at[pl.ds(row0, ROWS_PT)])
        plsc.subcore_barrier()
        for cc in range(NC):

            @pl.when(c == cc)
            def _(cc=cc):
                pltpu.sync_copy(src_hbm.at[s * 2 + cc], sidx)
                pltpu.sync_copy(dst_hbm.at[s * 2 + cc], didx)
                _gather_scatter_pipelined(g_hbm, sidx, didx, accum,
                                          r0, r1, sem0, sem1, 0, HALF_I)

        plsc.subcore_barrier()
        for cc, oref in enumerate((o0, o1)):

            @pl.when(c == cc)
            def _(oref=oref):
                pltpu.sync_copy(accum.at[pl.ds(row0, ROWS_PT)],
                                oref.at[pl.ds(row0, ROWS_PT)])

    return aggs(g, src3, dst3, zeros_hbm)


def _sc_edge_gather(z, src3, dst3):
    zcw = z.shape[1]
    iters = src3.shape[1]
    ept = iters * CHK
    epad = NS * ept
    out_type = [jax.ShapeDtypeStruct((epad, zcw), _f32)] * 2

    @functools.partial(
        pl.kernel,
        out_type=out_type,
        mesh=plsc.VectorSubcoreMesh(**_MESH),
        scratch_types=[
            pltpu.VMEM((iters, CHK), jnp.int32),
            pltpu.VMEM((CHK, zcw), _f32),
            pltpu.VMEM((CHK, zcw), _f32),
            pltpu.VMEM_SHARED((NPAD, zcw), _f32),
            pltpu.SemaphoreType.DMA,
            pltpu.SemaphoreType.DMA,
        ],
    )
    def gather_kernel(z_hbm, src_hbm, dst_hbm, es, ed, idx, r0, r1, zbuf,
                      sem0, sem1):
        c = lax.axis_index("c")
        s = lax.axis_index("s")
        row0 = s * ROWS_PT
        pltpu.sync_copy(z_hbm.at[pl.ds(row0, ROWS_PT)],
                        zbuf.at[pl.ds(row0, ROWS_PT)])
        plsc.subcore_barrier()
        for cc, (idx_hbm, oref) in enumerate(((src_hbm, es), (dst_hbm, ed))):

            @pl.when(c == cc)
            def _(idx_hbm=idx_hbm, oref=oref):
                pltpu.sync_copy(idx_hbm.at[s], idx)
                pltpu.async_copy(zbuf.at[idx.at[0]], r0, sem0)

                def body(i2, carry):
                    it = 2 * i2
                    ebase = s * ept + it * CHK
                    pltpu.async_copy(zbuf.at[idx.at[it + 1]], r1, sem1)
                    pltpu.make_async_copy(zbuf.at[idx.at[it]], r0,
                                          sem0).wait()
                    pltpu.sync_copy(r0, oref.at[pl.ds(ebase, CHK)])

                    @pl.when(it + 2 < iters)
                    def _():
                        pltpu.async_copy(zbuf.at[idx.at[it + 2]], r0, sem0)

                    pltpu.make_async_copy(zbuf.at[idx.at[it + 1]], r1,
                                          sem1).wait()
                    pltpu.sync_copy(r1, oref.at[pl.ds(ebase + CHK, CHK)])
                    return carry

                lax.fori_loop(0, iters // 2, body, 0)

    return gather_kernel(z, src3, dst3)


BM = 1024
BME = 2048


def _dinv_block(deg_ref, i, bm):
    deg = deg_ref[...][:, :1] + 1.0
    dinv = lax.rsqrt(deg)
    rows = i * bm + lax.broadcasted_iota(jnp.int32, (bm, 1), 0)
    return jnp.where(rows < N_NODES, dinv, 0.0)


def _tc_scale(x, deg16, cw):
    nout = x.shape[1] // cw

    def kern(x_ref, deg_ref, *out_refs):
        i = pl.program_id(0)
        dinv = _dinv_block(deg_ref, i, BM)
        t = x_ref[...] * dinv
        for k, o in enumerate(out_refs):
            o[...] = t[:, k * cw:(k + 1) * cw]

    return pl.pallas_call(
        kern,
        grid=(NPAD // BM,),
        in_specs=[
            pl.BlockSpec((BM, x.shape[1]), lambda i: (i, 0)),
            pl.BlockSpec((BM, DEGW), lambda i: (i, 0)),
        ],
        out_specs=[pl.BlockSpec((BM, cw), lambda i: (i, 0))] * nout,
        out_shape=[jax.ShapeDtypeStruct((NPAD, cw), _f32)] * nout,
    )(x, deg16)


def _tc_layer12(aggt, ts, b1, w1, w2, deg16, cw_out):
    nin = len(aggt)
    cwin = aggt[0].shape[1]
    nout = w2.shape[1] // cw_out

    def kern(*refs):
        agg_refs = refs[:nin]
        t_refs = refs[nin:2 * nin]
        b_ref, w1_ref, w2_ref, deg_ref = refs[2 * nin:2 * nin + 4]
        out_refs = refs[2 * nin + 4:]
        i = pl.program_id(0)
        dinv = _dinv_block(deg_ref, i, BM)
        parts = [agg_refs[k][...] + t_refs[k][...] for k in range(nin)]
        s = jnp.concatenate(parts, axis=1)
        h = jnp.dot(s, w1_ref[...], preferred_element_type=_f32)
        x2 = jnp.maximum(h * dinv + b_ref[...], 0.0)
        g = jnp.dot(x2, w2_ref[...], preferred_element_type=_f32) * dinv
        for k, o in enumerate(out_refs):
            o[...] = g[:, k * cw_out:(k + 1) * cw_out]

    return pl.pallas_call(
        kern,
        grid=(NPAD // BM,),
        in_specs=(
            [pl.BlockSpec((BM, cwin), lambda i: (i, 0))] * (2 * nin)
            + [
                pl.BlockSpec(b1.shape, lambda i: (0, 0)),
                pl.BlockSpec(w1.shape, lambda i: (0, 0)),
                pl.BlockSpec(w2.shape, lambda i: (0, 0)),
                pl.BlockSpec((BM, DEGW), lambda i: (i, 0)),
            ]
        ),
        out_specs=[pl.BlockSpec((BM, cw_out), lambda i: (i, 0))] * nout,
        out_shape=[jax.ShapeDtypeStruct((NPAD, cw_out), _f32)] * nout,
    )(*aggt, *ts, b1, w1, w2, deg16)


def _tc_layer_next(aggs, gs, b, w, deg16, cw_out):
    nin = len(aggs)
    cwin = aggs[0].shape[1]
    nout = w.shape[1] // cw_out

    def kern(*refs):
        agg_refs = refs[:nin]
        g_refs = refs[nin:2 * nin]
        b_ref, w_ref, deg_ref = refs[2 * nin:2 * nin + 3]
        out_refs = refs[2 * nin + 3:]
        i = pl.program_id(0)
        dinv = _dinv_block(deg_ref, i, BM)
        parts = [agg_refs[k][...] + g_refs[k][...] for k in range(nin)]
        x = jnp.concatenate(parts, axis=1) * dinv + b_ref[...]
        x = jnp.maximum(x, 0.0)
        g = jnp.dot(x, w_ref[...], preferred_element_type=_f32) * dinv
        for k, o in enumerate(out_refs):
            o[...] = g[:, k * cw_out:(k + 1) * cw_out]

    return pl.pallas_call(
        kern,
        grid=(NPAD // BM,),
        in_specs=(
            [pl.BlockSpec((BM, cwin), lambda i: (i, 0))] * (2 * nin)
            + [
                pl.BlockSpec(b.shape, lambda i: (0, 0)),
                pl.BlockSpec(w.shape, lambda i: (0, 0)),
                pl.BlockSpec((BM, DEGW), lambda i: (i, 0)),
            ]
        ),
        out_specs=[pl.BlockSpec((BM, cw_out), lambda i: (i, 0))] * nout,
        out_shape=[jax.ShapeDtypeStruct((NPAD, cw_out), _f32)] * nout,
    )(*aggs, *gs, b, w, deg16)


def _tc_z(p0, p1, g, b, deg16):
    cw = g.shape[1]

    def kern(p0_ref, p1_ref, g_ref, b_ref, deg_ref, out_ref):
        i = pl.program_id(0)
        dinv = _dinv_block(deg_ref, i, BM)
        z = (p0_ref[...] + p1_ref[...] + g_ref[...]) * dinv + b_ref[...]
        out_ref[...] = z

    return pl.pallas_call(
        kern,
        grid=(NPAD // BM,),
        in_specs=[
            pl.BlockSpec((BM, cw), lambda i: (i, 0)),
            pl.BlockSpec((BM, cw), lambda i: (i, 0)),
            pl.BlockSpec((BM, cw), lambda i: (i, 0)),
            pl.BlockSpec(b.shape, lambda i: (0, 0)),
            pl.BlockSpec((BM, DEGW), lambda i: (i, 0)),
        ],
        out_specs=pl.BlockSpec((BM, cw), lambda i: (i, 0)),
        out_shape=jax.ShapeDtypeStruct((NPAD, cw), _f32),
    )(p0, p1, g, b, deg16)


def _tc_decoder(es, ed, w1, b1, w2, b2, w3row, b3):
    zcw = es.shape[1]

    def kern(e0, e1, w1r, b1r, w2r, b2r, w3r, b3r, out_ref):
        e = jnp.concatenate([e0[...], e1[...]], axis=1)
        x = jnp.dot(e, w1r[...], preferred_element_type=_f32) + b1r[...]
        x = jnp.maximum(x, 0.0)
        x = jnp.dot(x, w2r[...], preferred_element_type=_f32) + b2r[...]
        x = jnp.maximum(x, 0.0)
        s = jnp.sum(x * w3r[...], axis=1) + b3r[0, 0]
        out_ref[...] = jnp.maximum(s, 0.0) + jnp.log(1.0 + jnp.exp(-jnp.abs(s)))

    return pl.pallas_call(
        kern,
        grid=(es.shape[0] // BME,),
        in_specs=[
            pl.BlockSpec((BME, zcw), lambda i: (i, 0)),
            pl.BlockSpec((BME, zcw), lambda i: (i, 0)),
            pl.BlockSpec(w1.shape, lambda i: (0, 0)),
            pl.BlockSpec(b1.shape, lambda i: (0, 0)),
            pl.BlockSpec(w2.shape, lambda i: (0, 0)),
            pl.BlockSpec(b2.shape, lambda i: (0, 0)),
            pl.BlockSpec(w3row.shape, lambda i: (0, 0)),
            pl.BlockSpec(b3.shape, lambda i: (0, 0)),
        ],
        out_specs=pl.BlockSpec((BME,), lambda i: (i,)),
        out_shape=jax.ShapeDtypeStruct((es.shape[0],), _f32),
    )(es, ed, w1, b1, w2, b2, w3row, b3)


def kernel(edge_features, edge_index, W1, b1, W2, b2, W3, b3,
           fc1_W, fc1_b, fc2_W, fc2_b, fc3_W, fc3_b):
    src = edge_index[0]
    dst = edge_index[1]
    pad_idx = jnp.full((EPAD - N_EDGES,), N_NODES, dtype=jnp.int32)
    srcp = jnp.concatenate([src, pad_idx])
    dstp = jnp.concatenate([dst, pad_idx])
    src3 = srcp.reshape(NS, ITERS, CHK)
    dst3 = dstp.reshape(NS, ITERS, CHK)
    src3a = srcp.reshape(NS * 2, HALF_I, CHKA)
    dst3a = dstp.reshape(NS * 2, HALF_I, CHKA)
    xpad = jnp.pad(edge_features, ((0, NPAD - N_NODES), (0, 0)))
    ones128 = jnp.ones((CHK, DEGW), _f32)
    zeros128 = jnp.zeros((NPAD, 128), _f32)

    deg16 = _sc_degree(dst3, ones128, zeros128)

    t1 = _tc_scale(xpad, deg16, 128)
    at = _sc_aggregate(t1, src3a, dst3a, zeros128, 128)
    g2 = _tc_layer12(at, t1, b1.reshape(1, H), W1, W2, deg16, 128)
    a2 = _sc_aggregate(g2, src3a, dst3a, zeros128, 128)
    (g3,) = _tc_layer_next(a2, g2, b2.reshape(1, H), W3, deg16, 128)
    p0, p1 = _sc_aggregate_split(g3, src3a, dst3a, zeros128)
    z = _tc_z(p0, p1, g3, b3.reshape(1, BOTTLE), deg16)

    dec_w = (fc1_W, fc1_b.reshape(1, DEC_H),
             fc2_W, fc2_b.reshape(1, DEC_H // 2),
             fc3_W.reshape(1, DEC_H // 2), fc3_b.reshape(1, 1))
    es, ed = _sc_edge_gather(z, src3, dst3)
    dec = _tc_decoder(es, ed, *dec_w)
    return dec[:N_EDGES]


# speedup vs baseline: 1.0383x; 1.0383x over previous
import functools

import jax
import jax.numpy as jnp
from jax import lax
from jax.experimental import pallas as pl
from jax.experimental.pallas import tpu as pltpu
from jax.experimental.pallas import tpu_sc as plsc

N_NODES = 10000
N_EDGES = 160000
NPAD = 10240
EPAD = 163840
D_IN, H, BOTTLE, DEC_H = 256, 512, 128, 512

NC, NS = 2, 16
CHK = 128
EPT = EPAD // NS
ITERS = EPT // CHK
CHKA = 64
ITERSA = EPT // CHKA
HALF_I = ITERSA // 2
ROWS_PT = NPAD // NS

_f32 = jnp.float32

_MESH = dict(core_axis_name="c", subcore_axis_name="s")


DEGW = 128


def _sc_degree(dst3, ones_hbm, zeros_hbm):
    @functools.partial(
        pl.kernel,
        out_type=jax.ShapeDtypeStruct((NPAD, DEGW), _f32),
        mesh=plsc.VectorSubcoreMesh(**_MESH),
        scratch_types=[
            pltpu.VMEM((ITERS, CHK), jnp.int32),
            pltpu.VMEM((CHK, DEGW), _f32),
            pltpu.VMEM_SHARED((NPAD, DEGW), _f32),
        ],
    )
    def deg_kernel(dst_hbm, ones_in, zeros_in, out_hbm, didx, ones_v, accum):
        c = lax.axis_index("c")
        s = lax.axis_index("s")
        row0 = s * ROWS_PT

        @pl.when(c == 0)
        def _():
            pltpu.sync_copy(zeros_in.at[pl.ds(row0, ROWS_PT)],
                            accum.at[pl.ds(row0, ROWS_PT)])
            pltpu.sync_copy(dst_hbm.at[s], didx)
            pltpu.sync_copy(ones_in, ones_v)
            plsc.subcore_barrier()

            def body(it, carry):
                pltpu.sync_copy(ones_v, accum.at[didx.at[it]], add=True)
                return carry

            lax.fori_loop(0, ITERS, body, 0)
            plsc.subcore_barrier()
            pltpu.sync_copy(accum.at[pl.ds(row0, ROWS_PT)],
                            out_hbm.at[pl.ds(row0, ROWS_PT)])

    return deg_kernel(dst3, ones_hbm, zeros_hbm)


def _gather_scatter_pipelined(g_ref, sidx, didx, accum, r0, r1, sem0, sem1,
                              base, iters):
    pltpu.async_copy(g_ref.at[sidx.at[base]], r0, sem0)

    def body(i2, carry):
        it = base + 2 * i2
        pltpu.async_copy(g_ref.at[sidx.at[it + 1]], r1, sem1)
        pltpu.make_async_copy(g_ref.at[sidx.at[it]], r0, sem0).wait()
        pltpu.sync_copy(r0, accum.at[didx.at[it]], add=True)

        @pl.when(2 * i2 + 2 < iters)
        def _():
            pltpu.async_copy(g_ref.at[sidx.at[it + 2]], r0, sem0)

        pltpu.make_async_copy(g_ref.at[sidx.at[it + 1]], r1, sem1).wait()
        pltpu.sync_copy(r1, accum.at[didx.at[it + 1]], add=True)
        return carry

    lax.fori_loop(0, iters // 2, body, 0)


def _agg_body(g_refs, src_hbm, dst_hbm, zeros_hbm, out_refs,
              sidx, didx, r0, r1, accum, sem0, sem1, cps):
    c = lax.axis_index("c")
    s = lax.axis_index("s")
    row0 = s * ROWS_PT
    for j in range(cps):
        pltpu.sync_copy(zeros_hbm.at[pl.ds(row0, ROWS_PT)],
                        accum.at[pl.ds(row0, ROWS_PT)])
        plsc.subcore_barrier()
        for cc in range(NC):
            cid = cc * cps + j

            @pl.when(c == cc)
            def _(cid=cid):
                for stage in range(2):
                    pltpu.sync_copy(src_hbm.at[s * 2 + stage], sidx)
                    pltpu.sync_copy(dst_hbm.at[s * 2 + stage], didx)
                    _gather_scatter_pipelined(g_refs[cid], sidx, didx, accum,
                                              r0, r1, sem0, sem1, 0, HALF_I)

        plsc.subcore_barrier()
        for cc in range(NC):
            cid = cc * cps + j

            @pl.when(c == cc)
            def _(cid=cid):
                pltpu.sync_copy(accum.at[pl.ds(row0, ROWS_PT)],
                                out_refs[cid].at[pl.ds(row0, ROWS_PT)])


def _sc_aggregate(g_chunks, src3, dst3, zeros_hbm, cw):
    nchunks = len(g_chunks)
    cps = nchunks // NC
    out_type = [jax.ShapeDtypeStruct((NPAD, cw), _f32)] * nchunks
    scratch = [
        pltpu.VMEM((HALF_I, CHKA), jnp.int32),
        pltpu.VMEM((HALF_I, CHKA), jnp.int32),
        pltpu.VMEM((CHKA, cw), _f32),
        pltpu.VMEM((CHKA, cw), _f32),
        pltpu.VMEM_SHARED((NPAD, cw), _f32),
        pltpu.SemaphoreType.DMA,
        pltpu.SemaphoreType.DMA,
    ]
    mesh = plsc.VectorSubcoreMesh(**_MESH)

    if nchunks == 4:
        @functools.partial(pl.kernel, out_type=out_type, mesh=mesh,
                           scratch_types=scratch)
        def agg4(g0, g1, g2, g3, src_hbm, dst_hbm, zin, o0, o1, o2, o3,
                 sidx, didx, r0, r1, accum, sem0, sem1):
            _agg_body((g0, g1, g2, g3), src_hbm, dst_hbm, zin,
                      (o0, o1, o2, o3), sidx, didx, r0, r1, accum,
                      sem0, sem1, cps)

        return agg4(*g_chunks, src3, dst3, zeros_hbm)

    @functools.partial(pl.kernel, out_type=out_type, mesh=mesh,
                       scratch_types=scratch)
    def agg2(g0, g1, src_hbm, dst_hbm, zin, o0, o1,
             sidx, didx, r0, r1, accum, sem0, sem1):
        _agg_body((g0, g1), src_hbm, dst_hbm, zin, (o0, o1),
                  sidx, didx, r0, r1, accum, sem0, sem1, cps)

    return agg2(*g_chunks, src3, dst3, zeros_hbm)


def _sc_aggregate_split(g, src3, dst3, zeros_hbm):
    cw = g.shape[1]
    out_type = [jax.ShapeDtypeStruct((NPAD, cw), _f32)] * NC

    @functools.partial(
        pl.kernel,
        out_type=out_type,
        mesh=plsc.VectorSubcoreMesh(**_MESH),
        scratch_types=[
            pltpu.VMEM((HALF_I, CHKA), jnp.int32),
            pltpu.VMEM((HALF_I, CHKA), jnp.int32),
            pltpu.VMEM((CHKA, cw), _f32),
            pltpu.VMEM((CHKA, cw), _f32),
            pltpu.VMEM_SHARED((NPAD, cw), _f32),
            pltpu.SemaphoreType.DMA,
            pltpu.SemaphoreType.DMA,
        ],
    )
    def aggs(g_hbm, src_hbm, dst_hbm, zin, o0, o1,
             sidx, didx, r0, r1, accum, sem0, sem1):
        c = lax.axis_index("c")
        s = lax.axis_index("s")
        row0 = s * ROWS_PT
        pltpu.sync_copy(zin.at[pl.ds(row0, ROWS_PT)],
                        accum.at[pl.ds(row0, ROWS_PT)])
        plsc.subcore_barrier()
        for cc in range(NC):

            @pl.when(c == cc)
            def _(cc=cc):
                pltpu.sync_copy(src_hbm.at[s * 2 + cc], sidx)
                pltpu.sync_copy(dst_hbm.at[s * 2 + cc], didx)
                _gather_scatter_pipelined(g_hbm, sidx, didx, accum,
                                          r0, r1, sem0, sem1, 0, HALF_I)

        plsc.subcore_barrier()
        for cc, oref in enumerate((o0, o1)):

            @pl.when(c == cc)
            def _(oref=oref):
                pltpu.sync_copy(accum.at[pl.ds(row0, ROWS_PT)],
                                oref.at[pl.ds(row0, ROWS_PT)])

    return aggs(g, src3, dst3, zeros_hbm)


def _sc_edge_gather(z, src3, dst3):
    zcw = z.shape[1]
    iters = src3.shape[1]
    ept = iters * CHK
    epad = NS * ept
    out_type = [jax.ShapeDtypeStruct((epad, zcw), _f32)] * 2

    @functools.partial(
        pl.kernel,
        out_type=out_type,
        mesh=plsc.VectorSubcoreMesh(**_MESH),
        scratch_types=[
            pltpu.VMEM((iters, CHK), jnp.int32),
            pltpu.VMEM((CHK, zcw), _f32),
            pltpu.VMEM((CHK, zcw), _f32),
            pltpu.VMEM_SHARED((NPAD, zcw), _f32),
            pltpu.SemaphoreType.DMA,
            pltpu.SemaphoreType.DMA,
        ],
    )
    def gather_kernel(z_hbm, src_hbm, dst_hbm, es, ed, idx, r0, r1, zbuf,
                      sem0, sem1):
        c = lax.axis_index("c")
        s = lax.axis_index("s")
        row0 = s * ROWS_PT
        pltpu.sync_copy(z_hbm.at[pl.ds(row0, ROWS_PT)],
                        zbuf.at[pl.ds(row0, ROWS_PT)])
        plsc.subcore_barrier()
        for cc, (idx_hbm, oref) in enumerate(((src_hbm, es), (dst_hbm, ed))):

            @pl.when(c == cc)
            def _(idx_hbm=idx_hbm, oref=oref):
                pltpu.sync_copy(idx_hbm.at[s], idx)
                pltpu.async_copy(zbuf.at[idx.at[0]], r0, sem0)

                def body(i2, carry):
                    it = 2 * i2
                    ebase = s * ept + it * CHK
                    pltpu.async_copy(zbuf.at[idx.at[it + 1]], r1, sem1)
                    pltpu.make_async_copy(zbuf.at[idx.at[it]], r0,
                                          sem0).wait()
                    pltpu.sync_copy(r0, oref.at[pl.ds(ebase, CHK)])

                    @pl.when(it + 2 < iters)
                    def _():
                        pltpu.async_copy(zbuf.at[idx.at[it + 2]], r0, sem0)

                    pltpu.make_async_copy(zbuf.at[idx.at[it + 1]], r1,
                                          sem1).wait()
                    pltpu.sync_copy(r1, oref.at[pl.ds(ebase + CHK, CHK)])
                    return carry

                lax.fori_loop(0, iters // 2, body, 0)

    return gather_kernel(z, src3, dst3)


BM = 1024
BME = 4096


def _dinv_block(deg_ref, i, bm):
    deg = deg_ref[...][:, :1] + 1.0
    dinv = lax.rsqrt(deg)
    rows = i * bm + lax.broadcasted_iota(jnp.int32, (bm, 1), 0)
    return jnp.where(rows < N_NODES, dinv, 0.0)


def _tc_scale(x, deg16, cw):
    nout = x.shape[1] // cw

    def kern(x_ref, deg_ref, *out_refs):
        i = pl.program_id(0)
        dinv = _dinv_block(deg_ref, i, BM)
        t = x_ref[...] * dinv
        for k, o in enumerate(out_refs):
            o[...] = t[:, k * cw:(k + 1) * cw]

    return pl.pallas_call(
        kern,
        grid=(NPAD // BM,),
        in_specs=[
            pl.BlockSpec((BM, x.shape[1]), lambda i: (i, 0)),
            pl.BlockSpec((BM, DEGW), lambda i: (i, 0)),
        ],
        out_specs=[pl.BlockSpec((BM, cw), lambda i: (i, 0))] * nout,
        out_shape=[jax.ShapeDtypeStruct((NPAD, cw), _f32)] * nout,
    )(x, deg16)


def _tc_layer12(aggt, ts, b1, w1, w2, deg16, cw_out):
    nin = len(aggt)
    cwin = aggt[0].shape[1]
    nout = w2.shape[1] // cw_out

    def kern(*refs):
        agg_refs = refs[:nin]
        t_refs = refs[nin:2 * nin]
        b_ref, w1_ref, w2_ref, deg_ref = refs[2 * nin:2 * nin + 4]
        out_refs = refs[2 * nin + 4:]
        i = pl.program_id(0)
        dinv = _dinv_block(deg_ref, i, BM)
        parts = [agg_refs[k][...] + t_refs[k][...] for k in range(nin)]
        s = jnp.concatenate(parts, axis=1)
        h = jnp.dot(s, w1_ref[...], preferred_element_type=_f32)
        x2 = jnp.maximum(h * dinv + b_ref[...], 0.0)
        g = jnp.dot(x2, w2_ref[...], preferred_element_type=_f32) * dinv
        for k, o in enumerate(out_refs):
            o[...] = g[:, k * cw_out:(k + 1) * cw_out]

    return pl.pallas_call(
        kern,
        grid=(NPAD // BM,),
        in_specs=(
            [pl.BlockSpec((BM, cwin), lambda i: (i, 0))] * (2 * nin)
            + [
                pl.BlockSpec(b1.shape, lambda i: (0, 0)),
                pl.BlockSpec(w1.shape, lambda i: (0, 0)),
                pl.BlockSpec(w2.shape, lambda i: (0, 0)),
                pl.BlockSpec((BM, DEGW), lambda i: (i, 0)),
            ]
        ),
        out_specs=[pl.BlockSpec((BM, cw_out), lambda i: (i, 0))] * nout,
        out_shape=[jax.ShapeDtypeStruct((NPAD, cw_out), _f32)] * nout,
    )(*aggt, *ts, b1, w1, w2, deg16)


def _tc_layer_next(aggs, gs, b, w, deg16, cw_out):
    nin = len(aggs)
    cwin = aggs[0].shape[1]
    nout = w.shape[1] // cw_out

    def kern(*refs):
        agg_refs = refs[:nin]
        g_refs = refs[nin:2 * nin]
        b_ref, w_ref, deg_ref = refs[2 * nin:2 * nin + 3]
        out_refs = refs[2 * nin + 3:]
        i = pl.program_id(0)
        dinv = _dinv_block(deg_ref, i, BM)
        parts = [agg_refs[k][...] + g_refs[k][...] for k in range(nin)]
        x = jnp.concatenate(parts, axis=1) * dinv + b_ref[...]
        x = jnp.maximum(x, 0.0)
        g = jnp.dot(x, w_ref[...], preferred_element_type=_f32) * dinv
        for k, o in enumerate(out_refs):
            o[...] = g[:, k * cw_out:(k + 1) * cw_out]

    return pl.pallas_call(
        kern,
        grid=(NPAD // BM,),
        in_specs=(
            [pl.BlockSpec((BM, cwin), lambda i: (i, 0))] * (2 * nin)
            + [
                pl.BlockSpec(b.shape, lambda i: (0, 0)),
                pl.BlockSpec(w.shape, lambda i: (0, 0)),
                pl.BlockSpec((BM, DEGW), lambda i: (i, 0)),
            ]
        ),
        out_specs=[pl.BlockSpec((BM, cw_out), lambda i: (i, 0))] * nout,
        out_shape=[jax.ShapeDtypeStruct((NPAD, cw_out), _f32)] * nout,
    )(*aggs, *gs, b, w, deg16)


def _tc_z(p0, p1, g, b, deg16):
    cw = g.shape[1]

    def kern(p0_ref, p1_ref, g_ref, b_ref, deg_ref, out_ref):
        i = pl.program_id(0)
        dinv = _dinv_block(deg_ref, i, BM)
        z = (p0_ref[...] + p1_ref[...] + g_ref[...]) * dinv + b_ref[...]
        out_ref[...] = z

    return pl.pallas_call(
        kern,
        grid=(NPAD // BM,),
        in_specs=[
            pl.BlockSpec((BM, cw), lambda i: (i, 0)),
            pl.BlockSpec((BM, cw), lambda i: (i, 0)),
            pl.BlockSpec((BM, cw), lambda i: (i, 0)),
            pl.BlockSpec(b.shape, lambda i: (0, 0)),
            pl.BlockSpec((BM, DEGW), lambda i: (i, 0)),
        ],
        out_specs=pl.BlockSpec((BM, cw), lambda i: (i, 0)),
        out_shape=jax.ShapeDtypeStruct((NPAD, cw), _f32),
    )(p0, p1, g, b, deg16)


def _tc_decoder(es, ed, w1, b1, w2, b2, w3row, b3):
    zcw = es.shape[1]

    def kern(e0, e1, w1r, b1r, w2r, b2r, w3r, b3r, out_ref):
        e = jnp.concatenate([e0[...], e1[...]], axis=1)
        x = jnp.dot(e, w1r[...], preferred_element_type=_f32) + b1r[...]
        x = jnp.maximum(x, 0.0)
        x = jnp.dot(x, w2r[...], preferred_element_type=_f32) + b2r[...]
        x = jnp.maximum(x, 0.0)
        s = jnp.sum(x * w3r[...], axis=1, keepdims=True) + b3r[0, 0]
        sp = jnp.maximum(s, 0.0) + jnp.log(1.0 + jnp.exp(-jnp.abs(s)))
        out_ref[...] = sp.reshape(1, 1, BME)

    return pl.pallas_call(
        kern,
        grid=(es.shape[0] // BME,),
        in_specs=[
            pl.BlockSpec((BME, zcw), lambda i: (i, 0)),
            pl.BlockSpec((BME, zcw), lambda i: (i, 0)),
            pl.BlockSpec(w1.shape, lambda i: (0, 0)),
            pl.BlockSpec(b1.shape, lambda i: (0, 0)),
            pl.BlockSpec(w2.shape, lambda i: (0, 0)),
            pl.BlockSpec(b2.shape, lambda i: (0, 0)),
            pl.BlockSpec(w3row.shape, lambda i: (0, 0)),
            pl.BlockSpec(b3.shape, lambda i: (0, 0)),
        ],
        out_specs=pl.BlockSpec((1, 1, BME), lambda i: (i, 0, 0)),
        out_shape=jax.ShapeDtypeStruct((es.shape[0] // BME, 1, BME), _f32),
    )(es, ed, w1, b1, w2, b2, w3row, b3).reshape(es.shape[0])


def kernel(edge_features, edge_index, W1, b1, W2, b2, W3, b3,
           fc1_W, fc1_b, fc2_W, fc2_b, fc3_W, fc3_b):
    src = edge_index[0]
    dst = edge_index[1]
    pad_idx = jnp.full((EPAD - N_EDGES,), N_NODES, dtype=jnp.int32)
    srcp = jnp.concatenate([src, pad_idx])
    dstp = jnp.concatenate([dst, pad_idx])
    src3 = srcp.reshape(NS, ITERS, CHK)
    dst3 = dstp.reshape(NS, ITERS, CHK)
    src3a = srcp.reshape(NS * 2, HALF_I, CHKA)
    dst3a = dstp.reshape(NS * 2, HALF_I, CHKA)
    xpad = jnp.pad(edge_features, ((0, NPAD - N_NODES), (0, 0)))
    ones128 = jnp.ones((CHK, DEGW), _f32)
    zeros128 = jnp.zeros((NPAD, 128), _f32)

    deg16 = _sc_degree(dst3, ones128, zeros128)

    t1 = _tc_scale(xpad, deg16, 128)
    at = _sc_aggregate(t1, src3a, dst3a, zeros128, 128)
    g2 = _tc_layer12(at, t1, b1.reshape(1, H), W1, W2, deg16, 128)
    a2 = _sc_aggregate(g2, src3a, dst3a, zeros128, 128)
    (g3,) = _tc_layer_next(a2, g2, b2.reshape(1, H), W3, deg16, 128)
    p0, p1 = _sc_aggregate_split(g3, src3a, dst3a, zeros128)
    z = _tc_z(p0, p1, g3, b3.reshape(1, BOTTLE), deg16)

    dec_w = (fc1_W, fc1_b.reshape(1, DEC_H),
             fc2_W, fc2_b.reshape(1, DEC_H // 2),
             fc3_W.reshape(1, DEC_H // 2), fc3_b.reshape(1, 1))
    srch = srcp.reshape(2, NS, ITERS // 2, CHK)
    dsth = dstp.reshape(2, NS, ITERS // 2, CHK)
    dec_halves = []
    for hh in range(2):
        es, ed = _sc_edge_gather(z, srch[hh], dsth[hh])
        dec_halves.append(_tc_decoder(es, ed, *dec_w))
    dec = jnp.concatenate(dec_halves)
    return dec[:N_EDGES]
